# Initial kernel scaffold; baseline (speedup 1.0000x reference)
import functools

import jax
import jax.numpy as jnp
from jax import lax
from jax.experimental import pallas as pl
from jax.experimental.pallas import tpu as pltpu
from jax.experimental.pallas import tpu_sc as plsc

N_OUT = 50000
N_IN = 12500
B = 4
C = 128
NW = 32
ROWS_PER_W = (B * N_OUT) // NW
R = 125
CH = ROWS_PER_W // R
LANES = 16


def _body(x_hbm, idx_hbm, val_hbm, out_hbm, idx_v, val_v, g_v, o_v, sem):
    cid = lax.axis_index("c")
    sid = lax.axis_index("s")
    wid = sid * 2 + cid

    def chunk(ci, _):
        base = wid * ROWS_PER_W + ci * R
        pltpu.sync_copy(idx_hbm.at[wid, ci], idx_v)
        pltpu.sync_copy(val_hbm.at[wid, ci], val_v)
        cps = [
            pltpu.async_copy(x_hbm.at[idx_v.at[j]], g_v.at[j], sem)
            for j in range(3)
        ]
        for cp in cps:
            cp.wait()

        def row(i, _):
            w0 = val_v[0, i]
            w1 = val_v[1, i]
            w2 = val_v[2, i]
            for c in range(C // LANES):
                sl = pl.ds(c * LANES, LANES)
                o_v[i, sl] = (
                    g_v[0, i, sl] * w0 + g_v[1, i, sl] * w1 + g_v[2, i, sl] * w2
                )
            return _

        lax.fori_loop(0, R, row, 0)
        pltpu.sync_copy(o_v, out_hbm.at[pl.ds(base, R)])
        return _

    lax.fori_loop(0, CH, chunk, 0)


@jax.jit
def _run(x2, idx_resh, val_resh):
    mesh = plsc.VectorSubcoreMesh(core_axis_name="c", subcore_axis_name="s")
    f = pl.kernel(
        _body,
        out_type=jax.ShapeDtypeStruct((B * N_OUT, C), jnp.float32),
        mesh=mesh,
        scratch_types=[
            pltpu.VMEM((3, R), jnp.int32),
            pltpu.VMEM((3, R), jnp.float32),
            pltpu.VMEM((3, R, C), jnp.float32),
            pltpu.VMEM((R, C), jnp.float32),
            pltpu.SemaphoreType.DMA,
        ],
    )
    return f(x2, idx_resh, val_resh)


def kernel(x, index, value):
    col = index[1]
    idx_all = col.reshape(1, N_OUT, 3) + (
        jnp.arange(B, dtype=jnp.int32) * N_IN
    ).reshape(B, 1, 1)
    val_all = jnp.broadcast_to(value.reshape(1, N_OUT, 3), (B, N_OUT, 3))
    idx_resh = idx_all.reshape(NW, CH, R, 3).transpose(0, 1, 3, 2)
    val_resh = val_all.reshape(NW, CH, R, 3).transpose(0, 1, 3, 2)
    x2 = x.reshape(B * N_IN, C)
    out2 = _run(x2, idx_resh, val_resh)
    return out2.reshape(B, N_OUT, C)


# speedup vs baseline: 4.0855x; 4.0855x over previous
import jax
import jax.numpy as jnp
from jax import lax
from jax.experimental import pallas as pl
from jax.experimental.pallas import tpu as pltpu
from jax.experimental.pallas import tpu_sc as plsc

N_OUT = 50000
N_IN = 12500
B = 4
C = 128
NW = 32
R = 128
CH = 49
ROWS_PER_W = CH * R
PAD_ROWS = NW * ROWS_PER_W
LANES = 16


def _body(x_hbm, idx_hbm, val_hbm, out_hbm, idx_v, val_v, g_v, o_v, sem):
    cid = lax.axis_index("c")
    sid = lax.axis_index("s")
    wid = sid * 2 + cid

    def chunk(ci, _):
        base = wid * ROWS_PER_W + ci * R
        pltpu.sync_copy(idx_hbm.at[wid, ci], idx_v)
        pltpu.sync_copy(val_hbm.at[wid, ci], val_v)
        cps = [
            pltpu.async_copy(x_hbm.at[idx_v.at[j]], g_v.at[j], sem)
            for j in range(3)
        ]
        for cp in cps:
            cp.wait()

        def row(i, _):
            w0 = val_v[0, i, :]
            w1 = val_v[1, i, :]
            w2 = val_v[2, i, :]
            for c in range(C // LANES):
                sl = pl.ds(c * LANES, LANES)
                o_v[i, sl] = (
                    g_v[0, i, sl] * w0 + g_v[1, i, sl] * w1 + g_v[2, i, sl] * w2
                )
            return _

        lax.fori_loop(0, R, row, 0)
        pltpu.sync_copy(o_v, out_hbm.at[pl.ds(base, R)])
        return _

    lax.fori_loop(0, CH, chunk, 0)


@jax.jit
def _run(x2, idx_resh, val_resh):
    mesh = plsc.VectorSubcoreMesh(core_axis_name="c", subcore_axis_name="s")
    f = pl.kernel(
        _body,
        out_type=jax.ShapeDtypeStruct((PAD_ROWS, C), jnp.float32),
        mesh=mesh,
        scratch_types=[
            pltpu.VMEM((3, R), jnp.int32),
            pltpu.VMEM((3, R, LANES), jnp.float32),
            pltpu.VMEM((3, R, C), jnp.float32),
            pltpu.VMEM((R, C), jnp.float32),
            pltpu.SemaphoreType.DMA,
        ],
    )
    return f(x2, idx_resh, val_resh)


def kernel(x, index, value):
    col = index[1]
    idx_all = (
        col.reshape(1, N_OUT, 3)
        + (jnp.arange(B, dtype=jnp.int32) * N_IN).reshape(B, 1, 1)
    ).reshape(B * N_OUT, 3)
    val_all = jnp.broadcast_to(value.reshape(1, N_OUT, 3), (B, N_OUT, 3)).reshape(
        B * N_OUT, 3
    )
    pad = PAD_ROWS - B * N_OUT
    idx_pad = jnp.concatenate(
        [idx_all, jnp.zeros((pad, 3), jnp.int32)], axis=0
    )
    val_pad = jnp.concatenate(
        [val_all, jnp.zeros((pad, 3), jnp.float32)], axis=0
    )
    idx_resh = idx_pad.reshape(NW, CH, R, 3).transpose(0, 1, 3, 2)
    val_resh = jnp.broadcast_to(
        val_pad.reshape(NW, CH, R, 3).transpose(0, 1, 3, 2)[..., None],
        (NW, CH, 3, R, LANES),
    )
    x2 = x.reshape(B * N_IN, C)
    out2 = _run(x2, idx_resh, val_resh)
    return out2[: B * N_OUT].reshape(B, N_OUT, C)


# speedup vs baseline: 8.4855x; 2.0770x over previous
import jax
import jax.numpy as jnp
from jax import lax
from jax.experimental import pallas as pl
from jax.experimental.pallas import tpu as pltpu
from jax.experimental.pallas import tpu_sc as plsc

N_OUT = 50000
N_IN = 12500
B = 4
C = 128
NW = 32
R = 64
NCHUNK = (B * N_OUT) // R
CH_MAX = -(-NCHUNK // NW)
FULL_W = NCHUNK - (CH_MAX - 1) * NW
LANES = 16
GROUPS = R // LANES
CSL = C // LANES


def _body(x_hbm, idx_hbm, val_hbm, out_hbm,
          idxs_v, vals_v, g_v, o_v, gsem0, gsem1):
    cid = lax.axis_index("c")
    sid = lax.axis_index("s")
    wid = sid * 2 + cid
    nch = jnp.where(wid < FULL_W, CH_MAX, CH_MAX - 1)
    gsems = (gsem0, gsem1)

    pltpu.sync_copy(idx_hbm.at[wid], idxs_v)
    pltpu.sync_copy(val_hbm.at[wid], vals_v)

    def start_gather(t, buf):
        for j in range(3):
            pltpu.async_copy(
                x_hbm.at[idxs_v.at[pl.ds(t * 3 * R + j * R, R)]],
                g_v.at[buf, j],
                gsems[buf],
            )

    def wait_gather(t, buf):
        for j in range(3):
            pltpu.make_async_copy(
                x_hbm.at[idxs_v.at[pl.ds(t * 3 * R + j * R, R)]],
                g_v.at[buf, j],
                gsems[buf],
            ).wait()

    def compute_write(t, buf):
        def group(q, _):
            wv = [
                vals_v[pl.ds(t * 3 * R + j * R + q * LANES, LANES)]
                for j in range(3)
            ]
            for k in range(LANES):
                i = q * LANES + k
                w0, w1, w2 = wv[0][k], wv[1][k], wv[2][k]
                for c in range(CSL):
                    sl = pl.ds(c * LANES, LANES)
                    o_v[buf, i, sl] = (
                        g_v[buf, 0, i, sl] * w0
                        + g_v[buf, 1, i, sl] * w1
                        + g_v[buf, 2, i, sl] * w2
                    )
            return _

        lax.fori_loop(0, GROUPS, group, 0)
        base = (wid + t * NW) * R
        pltpu.sync_copy(o_v.at[buf], out_hbm.at[pl.ds(base, R)])

    start_gather(0, 0)

    def pair(p, _):
        for b in range(2):
            t = 2 * p + b
            tn = t + 1

            @pl.when(tn < nch)
            def _prefetch():
                start_gather(tn, 1 - b)

            @pl.when(t < nch)
            def _do():
                wait_gather(t, b)
                compute_write(t, b)
        return _

    lax.fori_loop(0, CH_MAX // 2, pair, 0)


@jax.jit
def _run(x2, idx_resh, val_resh):
    mesh = plsc.VectorSubcoreMesh(core_axis_name="c", subcore_axis_name="s")
    f = pl.kernel(
        _body,
        out_type=jax.ShapeDtypeStruct((B * N_OUT, C), jnp.float32),
        mesh=mesh,
        scratch_types=[
            pltpu.VMEM((CH_MAX * 3 * R,), jnp.int32),
            pltpu.VMEM((CH_MAX * 3 * R,), jnp.float32),
            pltpu.VMEM((2, 3, R, C), jnp.float32),
            pltpu.VMEM((2, R, C), jnp.float32),
            pltpu.SemaphoreType.DMA,
            pltpu.SemaphoreType.DMA,
        ],
    )
    return f(x2, idx_resh, val_resh)


def kernel(x, index, value):
    col = index[1]
    idx_all = (
        col.reshape(1, N_OUT, 3)
        + (jnp.arange(B, dtype=jnp.int32) * N_IN).reshape(B, 1, 1)
    ).reshape(B * N_OUT, 3)
    val_all = jnp.broadcast_to(value.reshape(1, N_OUT, 3), (B, N_OUT, 3)).reshape(
        B * N_OUT, 3
    )
    pad = CH_MAX * NW - NCHUNK

    def layout(a):
        a = a.reshape(NCHUNK, R, 3).transpose(0, 2, 1)
        a = jnp.concatenate(
            [a, jnp.zeros((pad,) + a.shape[1:], a.dtype)], axis=0
        )
        return a.reshape(CH_MAX, NW, 3, R).transpose(1, 0, 2, 3).reshape(
            NW, CH_MAX * 3 * R
        )

    x2 = x.reshape(B * N_IN, C)
    out2 = _run(x2, layout(idx_all), layout(val_all))
    return out2.reshape(B, N_OUT, C)
